# baseline (speedup 1.0000x reference)
import functools

import jax
import jax.numpy as jnp
from jax.experimental import pallas as pl
from jax.experimental.pallas import tpu as pltpu

N = 1024
D = 128
BK = 1024


def _nn_kernel(emb_ref, w_ref, b_ref, tab_ref, out_ref, best_ref, idx_ref,
               *, nsteps, blk):
    j = pl.program_id(0)

    @pl.when(j == 0)
    def _init():
        best_ref[...] = jnp.full((N, 1), jnp.inf, jnp.float32)
        idx_ref[...] = jnp.zeros((N, 1), jnp.int32)

    y = (emb_ref[...] - b_ref[...]) / (w_ref[...] + 1e-6)

    tb = tab_ref[...]
    ones_row = jnp.ones((1, D), jnp.float32)
    contract = (((1,), (1,)), ((), ()))
    t2 = jax.lax.dot_general(ones_row, tb * tb, contract,
                             precision=jax.lax.Precision.HIGHEST,
                             preferred_element_type=jnp.float32)
    mm = jax.lax.dot_general(y, tb, contract,
                             preferred_element_type=jnp.float32)
    s = t2 - 2.0 * mm

    local_min = jnp.min(s, axis=1, keepdims=True)
    local_arg = jnp.argmin(s, axis=1).astype(jnp.int32)
    local_idx = (j * blk + local_arg)[:, None]

    upd = local_min < best_ref[...]
    idx_ref[...] = jnp.where(upd, local_idx, idx_ref[...])
    best_ref[...] = jnp.where(upd, local_min, best_ref[...])

    @pl.when(j == nsteps - 1)
    def _done():
        out_ref[...] = idx_ref[...]


@jax.jit
def kernel(embeddings, ln_weight, ln_bias, table):
    vocab = table.shape[0]
    nsteps = pl.cdiv(vocab, BK)
    padded = nsteps * BK
    if padded != vocab:
        table = jnp.pad(table, ((0, padded - vocab), (0, 0)), mode="edge")

    out = pl.pallas_call(
        functools.partial(_nn_kernel, nsteps=nsteps, blk=BK),
        grid=(nsteps,),
        in_specs=[
            pl.BlockSpec((N, D), lambda j: (0, 0)),
            pl.BlockSpec((1, D), lambda j: (0, 0)),
            pl.BlockSpec((1, D), lambda j: (0, 0)),
            pl.BlockSpec((BK, D), lambda j: (j, 0)),
        ],
        out_specs=pl.BlockSpec((N, 1), lambda j: (0, 0)),
        out_shape=jax.ShapeDtypeStruct((N, 1), jnp.int32),
        scratch_shapes=[
            pltpu.VMEM((N, 1), jnp.float32),
            pltpu.VMEM((N, 1), jnp.int32),
        ],
    )(embeddings, ln_weight[None, :], ln_bias[None, :], table)
    return out[:, 0]


# speedup vs baseline: 1.3057x; 1.3057x over previous
import functools

import jax
import jax.numpy as jnp
from jax.experimental import pallas as pl
from jax.experimental.pallas import tpu as pltpu

N = 1024
D = 128
BK = 2048


def _nn_kernel(emb_ref, w_ref, b_ref, tab_ref, out_ref, best_ref, idx_ref,
               *, nsteps, blk):
    j = pl.program_id(0)

    y = (emb_ref[...] - b_ref[...]) / (w_ref[...] + 1e-6)

    tb = tab_ref[...]
    ones_row = jnp.ones((1, D), jnp.float32)
    contract = (((1,), (1,)), ((), ()))
    t2h = 0.5 * jax.lax.dot_general(ones_row, tb * tb, contract,
                                    precision=jax.lax.Precision.HIGHEST,
                                    preferred_element_type=jnp.float32)
    mm = jax.lax.dot_general(y, tb, contract,
                             preferred_element_type=jnp.float32)
    s = t2h - mm

    col = j * blk + jax.lax.broadcasted_iota(jnp.int32, (1, blk), 1)

    @pl.when(j == 0)
    def _init():
        best_ref[...] = s
        idx_ref[...] = jnp.broadcast_to(col, (N, blk))

    @pl.when(j > 0)
    def _fold():
        prev = best_ref[...]
        upd = s < prev
        best_ref[...] = jnp.where(upd, s, prev)
        idx_ref[...] = jnp.where(upd, jnp.broadcast_to(col, (N, blk)),
                                 idx_ref[...])

    @pl.when(j == nsteps - 1)
    def _done():
        m = best_ref[...]
        rowmin = jnp.min(m, axis=1, keepdims=True)
        big = jnp.int32(2147483647)
        cand = jnp.where(m == rowmin, idx_ref[...], big)
        out_ref[...] = jnp.min(cand, axis=1, keepdims=True)


@jax.jit
def kernel(embeddings, ln_weight, ln_bias, table):
    vocab = table.shape[0]
    nsteps = pl.cdiv(vocab, BK)
    padded = nsteps * BK
    if padded != vocab:
        table = jnp.pad(table, ((0, padded - vocab), (0, 0)), mode="edge")

    out = pl.pallas_call(
        functools.partial(_nn_kernel, nsteps=nsteps, blk=BK),
        grid=(nsteps,),
        in_specs=[
            pl.BlockSpec((N, D), lambda j: (0, 0)),
            pl.BlockSpec((1, D), lambda j: (0, 0)),
            pl.BlockSpec((1, D), lambda j: (0, 0)),
            pl.BlockSpec((BK, D), lambda j: (j, 0)),
        ],
        out_specs=pl.BlockSpec((N, 1), lambda j: (0, 0)),
        out_shape=jax.ShapeDtypeStruct((N, 1), jnp.int32),
        scratch_shapes=[
            pltpu.VMEM((N, BK), jnp.float32),
            pltpu.VMEM((N, BK), jnp.int32),
        ],
    )(embeddings, ln_weight[None, :], ln_bias[None, :], table)
    return out[:, 0]
